# baseline (speedup 1.0000x reference)
import jax
import jax.numpy as jnp
from jax import lax
from jax.experimental import pallas as pl
from jax.experimental.pallas import tpu as pltpu
from jax.experimental.pallas import tpu_sc as plsc

D_MODEL = 768
B = 4
S = 4096
EPS = 1e-12
NW = 32
P_PER_W = S // NW
G = 64
NJ = D_MODEL // 16
TUNROLL = 1
JUNROLL = 4


def _ln_chunk(rows_v, pos_v, g_v, b_v):
    @plsc.parallel_loop(0, G, 1, unroll=TUNROLL)
    def token_body(t):
        zero = jnp.zeros((16,), jnp.float32)

        @plsc.parallel_loop(0, NJ, 1, unroll=JUNROLL, carry=(zero, zero))
        def pass1(j, carry):
            acc, acc2 = carry
            sl = pl.ds(j * 16, 16)
            x = rows_v[t, sl] + pos_v[t, sl]
            rows_v[t, sl] = x
            return acc + x, acc2 + x * x

        acc, acc2 = pass1
        mean = jnp.sum(acc) * (1.0 / D_MODEL)
        var = jnp.sum(acc2) * (1.0 / D_MODEL) - mean * mean
        ones = jnp.ones((16,), jnp.float32)
        mean_v = mean * ones
        v = (var + EPS) * ones
        iv = plsc.bitcast(v, jnp.int32)
        iv = 0x5F3759DF - (iv >> 1)
        y = plsc.bitcast(iv, jnp.float32)
        half_v = 0.5 * v
        for _n in range(3):
            y = y * (1.5 - half_v * y * y)
        @plsc.parallel_loop(0, NJ, 1, unroll=JUNROLL)
        def pass2(j):
            sl = pl.ds(j * 16, 16)
            x = rows_v[t, sl]
            rows_v[t, sl] = (x - mean_v) * y * g_v[sl] + b_v[sl]


def _sc_body(ids_hbm, wt_hbm, pt_hbm, g_hbm, b_hbm, out_hbm,
             idx_v, rows_v, pos_v, g_v, b_v, sem):
    wid = lax.axis_index("s") * 2 + lax.axis_index("c")
    p0 = wid * P_PER_W
    pltpu.sync_copy(g_hbm, g_v)
    pltpu.sync_copy(b_hbm, b_v)

    def pc_body(pc, _):
        pbase = p0 + pc * G
        pltpu.sync_copy(pt_hbm.at[pl.ds(pbase, G)], pos_v)

        def b_body(bb, _):
            tok = bb * S + pbase
            pltpu.sync_copy(ids_hbm.at[pl.ds(tok, G)], idx_v)
            pltpu.async_copy(wt_hbm.at[idx_v], rows_v, sem).wait()
            _ln_chunk(rows_v, pos_v, g_v, b_v)
            pltpu.sync_copy(rows_v, out_hbm.at[pl.ds(tok, G)])
            return 0

        lax.fori_loop(0, B, b_body, 0)
        return 0

    lax.fori_loop(0, P_PER_W // G, pc_body, 0)


@jax.jit
def _run(ids_flat, word_table, pos_table, gamma, beta):
    mesh = plsc.VectorSubcoreMesh(core_axis_name="c", subcore_axis_name="s")
    k = pl.kernel(
        _sc_body,
        out_type=jax.ShapeDtypeStruct((B * S, D_MODEL), jnp.float32),
        mesh=mesh,
        compiler_params=pltpu.CompilerParams(needs_layout_passes=False),
        scratch_types=[
            pltpu.VMEM((G,), jnp.int32),
            pltpu.VMEM((G, D_MODEL), jnp.float32),
            pltpu.VMEM((G, D_MODEL), jnp.float32),
            pltpu.VMEM((D_MODEL,), jnp.float32),
            pltpu.VMEM((D_MODEL,), jnp.float32),
            pltpu.SemaphoreType.DMA,
        ],
    )
    return k(ids_flat, word_table, pos_table, gamma, beta)


def kernel(input_ids, word_table, pos_table, gamma, beta):
    ids_flat = jnp.reshape(input_ids.astype(jnp.int32), (B * S,))
    out = _run(ids_flat, word_table, pos_table, gamma, beta)
    return jnp.reshape(out, (B, S, D_MODEL))


# speedup vs baseline: 1.0674x; 1.0674x over previous
import jax
import jax.numpy as jnp
from jax import lax
from jax.experimental import pallas as pl
from jax.experimental.pallas import tpu as pltpu
from jax.experimental.pallas import tpu_sc as plsc

D_MODEL = 768
B = 4
S = 4096
EPS = 1e-12
NW = 32
P_PER_W = S // NW
G = 32
NCHUNK = (P_PER_W // G) * B
NJ = D_MODEL // 16


def _ln_tokens(rows_v, hbase, pos_v, pbase, g_v, b_v):
    @plsc.parallel_loop(0, G, 1, unroll=1)
    def token_body(t):
        r = hbase + t
        p = pbase + t
        acc = jnp.zeros((16,), jnp.float32)
        acc2 = jnp.zeros((16,), jnp.float32)
        for j in range(NJ):
            sl = pl.ds(j * 16, 16)
            x = rows_v[r, sl] + pos_v[p, sl]
            rows_v[r, sl] = x
            acc = acc + x
            acc2 = acc2 + x * x
        mean = jnp.sum(acc) * (1.0 / D_MODEL)
        var = jnp.sum(acc2) * (1.0 / D_MODEL) - mean * mean
        ones = jnp.ones((16,), jnp.float32)
        mean_v = mean * ones
        v = (var + EPS) * ones
        iv = plsc.bitcast(v, jnp.int32)
        iv = 0x5F3759DF - (iv >> 1)
        y = plsc.bitcast(iv, jnp.float32)
        half_v = 0.5 * v
        for _n in range(3):
            y = y * (1.5 - half_v * y * y)
        for j in range(NJ):
            sl = pl.ds(j * 16, 16)
            x = rows_v[r, sl]
            rows_v[r, sl] = (x - mean_v) * y * g_v[sl] + b_v[sl]


def _sc_body(ids_hbm, wt_hbm, pt_hbm, g_hbm, b_hbm, out_hbm,
             idx_all, rows_v, pos_v, g_v, b_v, gsem, ssem):
    wid = lax.axis_index("s") * 2 + lax.axis_index("c")
    p0 = wid * P_PER_W
    pltpu.sync_copy(g_hbm, g_v)
    pltpu.sync_copy(b_hbm, b_v)
    for bb in range(B):
        pltpu.sync_copy(ids_hbm.at[pl.ds(bb * S + p0, P_PER_W)],
                        idx_all.at[pl.ds(bb * P_PER_W, P_PER_W)])

    def idx_off(c):
        return (c % B) * P_PER_W + (c // B) * G

    def tok_of(c):
        return (c % B) * S + p0 + (c // B) * G

    def start_gather(c):
        half = (c % 2) * G
        src = wt_hbm.at[idx_all.at[pl.ds(idx_off(c), G)]]
        pltpu.make_async_copy(src, rows_v.at[pl.ds(half, G)], gsem).start()

    def wait_rows(sem):
        pltpu.make_async_copy(wt_hbm.at[idx_all.at[pl.ds(0, G)]],
                              rows_v.at[pl.ds(0, G)], sem).wait()

    def start_store(c):
        half = (c % 2) * G
        pltpu.make_async_copy(rows_v.at[pl.ds(half, G)],
                              out_hbm.at[pl.ds(tok_of(c), G)], ssem).start()

    start_gather(0)

    def chunk_body(c, _):
        hbase = (c % 2) * G

        @pl.when(c + 1 < NCHUNK)
        def _():
            @pl.when(c >= 1)
            def _():
                wait_rows(ssem)

            start_gather(c + 1)

        wait_rows(gsem)

        @pl.when(lax.rem(c, B) == 0)
        def _():
            pltpu.sync_copy(pt_hbm.at[pl.ds(p0 + (c // B) * G, G)], pos_v)

        _ln_tokens(rows_v, hbase, pos_v, 0, g_v, b_v)
        start_store(c)
        return 0

    lax.fori_loop(0, NCHUNK, chunk_body, 0)
    wait_rows(ssem)
    wait_rows(ssem)


@jax.jit
def _run(ids_flat, word_table, pos_table, gamma, beta):
    mesh = plsc.VectorSubcoreMesh(core_axis_name="c", subcore_axis_name="s")
    k = pl.kernel(
        _sc_body,
        out_type=jax.ShapeDtypeStruct((B * S, D_MODEL), jnp.float32),
        mesh=mesh,
        compiler_params=pltpu.CompilerParams(needs_layout_passes=False),
        scratch_types=[
            pltpu.VMEM((B * P_PER_W,), jnp.int32),
            pltpu.VMEM((2 * G, D_MODEL), jnp.float32),
            pltpu.VMEM((G, D_MODEL), jnp.float32),
            pltpu.VMEM((D_MODEL,), jnp.float32),
            pltpu.VMEM((D_MODEL,), jnp.float32),
            pltpu.SemaphoreType.DMA,
            pltpu.SemaphoreType.DMA,
        ],
    )
    return k(ids_flat, word_table, pos_table, gamma, beta)


def kernel(input_ids, word_table, pos_table, gamma, beta):
    ids_flat = jnp.reshape(input_ids.astype(jnp.int32), (B * S,))
    out = _run(ids_flat, word_table, pos_table, gamma, beta)
    return jnp.reshape(out, (B, S, D_MODEL))
